# baseline (speedup 1.0000x reference)
import jax
import jax.numpy as jnp
from jax import lax
from jax.experimental import pallas as pl
from jax.experimental.pallas import tpu as pltpu
from jax.experimental.pallas import tpu_sc as plsc

_N = 65536
_R = 10
_D = 128
_NC = 2
_NS = 16
_NW = _NC * _NS
_RPW = _N // _NW
_CB = 32
_NCHUNK = _RPW // _CB


def _sc_body(x_hbm, o_hbm, in_v, out_v):
    c = lax.axis_index("c")
    s = lax.axis_index("s")
    wid = s * _NC + c
    base = wid * _RPW

    def chunk(i, carry):
        off = base + i * _CB
        pltpu.sync_copy(x_hbm.at[pl.ds(off, _CB)], in_v)

        def row(b, carry2):
            for f in range(_D // 16):
                sl = pl.ds(f * 16, 16)
                p0 = (in_v[b, 0, sl] * in_v[b, 1, sl] * in_v[b, 2, sl]
                      * in_v[b, 3, sl] * in_v[b, 4, sl])
                p1 = (in_v[b, 5, sl] * in_v[b, 6, sl] * in_v[b, 7, sl]
                      * in_v[b, 8, sl] * in_v[b, 9, sl])
                out_v[b, 0, sl] = p0
                out_v[b, 1, sl] = p1
            return carry2

        lax.fori_loop(0, _CB, row, 0)
        pltpu.sync_copy(out_v, o_hbm.at[pl.ds(off, _CB)])
        return carry

    lax.fori_loop(0, _NCHUNK, chunk, 0)


def kernel(inputs):
    mesh = plsc.VectorSubcoreMesh(core_axis_name="c", subcore_axis_name="s")
    f = pl.kernel(
        _sc_body,
        mesh=mesh,
        out_type=jax.ShapeDtypeStruct((_N, 2, _D), jnp.float32),
        scratch_types=[
            pltpu.VMEM((_CB, _R, _D), jnp.float32),
            pltpu.VMEM((_CB, 2, _D), jnp.float32),
        ],
    )
    return f(inputs)


# speedup vs baseline: 1.3283x; 1.3283x over previous
import jax
import jax.numpy as jnp
from jax.experimental import pallas as pl
from jax.experimental.pallas import tpu as pltpu

_B = 2048


def _body(x_ref, o_ref):
    x = x_ref[...]
    p0 = x[:, 0, :] * x[:, 1, :] * x[:, 2, :] * x[:, 3, :] * x[:, 4, :]
    p1 = x[:, 5, :] * x[:, 6, :] * x[:, 7, :] * x[:, 8, :] * x[:, 9, :]
    o_ref[...] = jnp.stack([p0, p1], axis=1)


def kernel(inputs):
    n, r, d = inputs.shape
    grid = (n // _B,)
    return pl.pallas_call(
        _body,
        grid=grid,
        in_specs=[pl.BlockSpec((_B, r, d), lambda i: (i, 0, 0))],
        out_specs=pl.BlockSpec((_B, 2, d), lambda i: (i, 0, 0)),
        out_shape=jax.ShapeDtypeStruct((n, 2, d), inputs.dtype),
    )(inputs)
